# baseline (speedup 1.0000x reference)
import dataclasses
import functools

import jax
import jax.numpy as jnp
from jax import lax
from jax.experimental import pallas as pl
from jax.experimental.pallas import tpu as pltpu
from jax.experimental.pallas import tpu_sc as plsc

N = 50000
E = 800000
HID = 64
HEADS = 4
DOUT = 7

NC = 2
NT = 16
LANES = 16

NP = 50016
RPT = NP // NT
CH = 24
E2 = E + N
NCHUNK = 2214
EPT = NCHUNK * CH
EP = EPT * NT
ACCW = 40

_ROWS = 2000
_GRID = N // _ROWS


def _tc_encode_body(x_ref, w1, b1, w2, b2, wl, bl, wr, br, xl_ref, xr_ref):
    h = jnp.maximum(jnp.dot(x_ref[...], w1[...],
                            preferred_element_type=jnp.float32) + b1[...], 0.0)
    h = jnp.dot(h, w2[...], preferred_element_type=jnp.float32) + b2[...]
    xl_ref[...] = jnp.dot(h, wl[...], preferred_element_type=jnp.float32) + bl[...]
    xr_ref[...] = jnp.dot(h, wr[...], preferred_element_type=jnp.float32) + br[...]


def _norm_heads(a, b):
    h0 = a[:, 0:16] / (a[:, 32:33] + 1e-16)
    h1 = a[:, 16:32] / (a[:, 33:34] + 1e-16)
    h2 = b[:, 0:16] / (b[:, 32:33] + 1e-16)
    h3 = b[:, 16:32] / (b[:, 33:34] + 1e-16)
    return jnp.concatenate([h0, h1, h2, h3], axis=1)


def _tc_mid_body(a_ref, b_ref, bias, wl, bl, wr, br, xl_ref, xr_ref):
    h = jnp.maximum(_norm_heads(a_ref[...], b_ref[...]) + bias[...], 0.0)
    xl_ref[...] = jnp.dot(h, wl[...], preferred_element_type=jnp.float32) + bl[...]
    xr_ref[...] = jnp.dot(h, wr[...], preferred_element_type=jnp.float32) + br[...]


def _tc_final_body(a_ref, b_ref, bias, hs_ref, wih, bih, whh, bhh,
                   dw1, db1, dw2, db2, out_ref, nh_ref):
    h = _norm_heads(a_ref[...], b_ref[...]) + bias[...]
    hs = hs_ref[...]
    gi = jnp.dot(h, wih[...], preferred_element_type=jnp.float32) + bih[...]
    gh = jnp.dot(hs, whh[...], preferred_element_type=jnp.float32) + bhh[...]
    r = jax.nn.sigmoid(gi[:, 0:64] + gh[:, 0:64])
    z = jax.nn.sigmoid(gi[:, 64:128] + gh[:, 64:128])
    n = jnp.tanh(gi[:, 128:192] + r * gh[:, 128:192])
    nh = (1.0 - z) * n + z * hs
    d = jnp.maximum(jnp.dot(nh, dw1[...], preferred_element_type=jnp.float32)
                    + db1[...], 0.0)
    out_ref[...] = jnp.dot(d, dw2[...], preferred_element_type=jnp.float32) + db2[...]
    nh_ref[...] = nh


def _full(shape):
    return pl.BlockSpec(shape, lambda i: tuple(0 for _ in shape))


def _rows(width):
    return pl.BlockSpec((_ROWS, width), lambda i: (i, 0))


def _tc_encode(x, w1, b1, w2, b2, wl, bl, wr, br):
    return pl.pallas_call(
        _tc_encode_body,
        grid=(_GRID,),
        in_specs=[_rows(8), _full((8, 64)), _full((1, 64)), _full((64, 64)),
                  _full((1, 64)), _full((64, 64)), _full((1, 64)),
                  _full((64, 64)), _full((1, 64))],
        out_specs=[_rows(64), _rows(64)],
        out_shape=[jax.ShapeDtypeStruct((N, 64), jnp.float32),
                   jax.ShapeDtypeStruct((N, 64), jnp.float32)],
    )(x, w1, b1, w2, b2, wl, bl, wr, br)


def _tc_mid(a, b, bias, wl, bl, wr, br):
    return pl.pallas_call(
        _tc_mid_body,
        grid=(_GRID,),
        in_specs=[_rows(ACCW), _rows(ACCW), _full((1, 64)), _full((64, 64)),
                  _full((1, 64)), _full((64, 64)), _full((1, 64))],
        out_specs=[_rows(64), _rows(64)],
        out_shape=[jax.ShapeDtypeStruct((N, 64), jnp.float32),
                   jax.ShapeDtypeStruct((N, 64), jnp.float32)],
    )(a, b, bias, wl, bl, wr, br)


def _tc_final(a, b, bias, hs, wih, bih, whh, bhh, dw1, db1, dw2, db2):
    return pl.pallas_call(
        _tc_final_body,
        grid=(_GRID,),
        in_specs=[_rows(ACCW), _rows(ACCW), _full((1, 64)), _rows(64),
                  _full((64, 192)), _full((1, 192)), _full((64, 192)),
                  _full((1, 192)), _full((64, 64)), _full((1, 64)),
                  _full((64, 7)), _full((1, 7))],
        out_specs=[_rows(7), _rows(64)],
        out_shape=[jax.ShapeDtypeStruct((N, 7), jnp.float32),
                   jax.ShapeDtypeStruct((N, 64), jnp.float32)],
    )(a, b, bias, hs, wih, bih, whh, bhh, dw1, db1, dw2, db2)


def _sc_gat(xtab, rtab, att, soff, doff, dst, zat):
    mesh = plsc.VectorSubcoreMesh(core_axis_name="c", subcore_axis_name="s")
    cp = pltpu.CompilerParams()
    for fld, val in (("needs_layout_passes", False),
                     ("use_tc_tiling_on_sc", False)):
        if fld in pltpu.CompilerParams.__dataclass_fields__:
            cp = dataclasses.replace(cp, **{fld: val})

    @functools.partial(
        pl.kernel,
        mesh=mesh,
        compiler_params=cp,
        out_type=jax.ShapeDtypeStruct((NC, NP, ACCW), jnp.float32),
        scratch_types=[
            pltpu.VMEM((2, CH), jnp.int32),
            pltpu.VMEM((2, CH), jnp.int32),
            pltpu.VMEM((2, CH), jnp.int32),
            pltpu.VMEM((2, CH, 32), jnp.float32),
            pltpu.VMEM((2, CH, 32), jnp.float32),
            pltpu.VMEM((2, CH, ACCW), jnp.float32),
            pltpu.VMEM((2, 16), jnp.float32),
            pltpu.SemaphoreType.DMA,
            pltpu.SemaphoreType.DMA,
            pltpu.VMEM_SHARED((NP, ACCW), jnp.float32),
        ],
    )
    def k(xt_h, rt_h, att_h, soff_h, doff_h, dst_h, z_h, acc_out,
          dsti, srco, dsto, xlb, xrb, ob, attb, sem0, sem1, accs):
        c = lax.axis_index("c")
        s = lax.axis_index("s")
        pltpu.sync_copy(att_h.at[pl.ds(2 * c, 2)], attb)
        pltpu.sync_copy(z_h, accs.at[pl.ds(s * RPT, RPT)])
        plsc.subcore_barrier()
        lid = lax.iota(jnp.int32, 16)
        att0 = attb[0]
        att1 = attb[1]
        sems = (sem0, sem1)
        cbase = s * NCHUNK

        def fire(b, kk):
            base = (cbase + kk) * CH
            pltpu.sync_copy(soff_h.at[c, pl.ds(base, CH)], srco.at[b])
            pltpu.sync_copy(doff_h.at[c, pl.ds(base, CH)], dsto.at[b])
            pltpu.sync_copy(dst_h.at[pl.ds(base, CH)], dsti.at[b])
            pltpu.async_copy(xt_h.at[srco.at[b]], xlb.at[b], sems[b])
            pltpu.async_copy(rt_h.at[dsto.at[b]], xrb.at[b], sems[b])

        def drain(b):
            pltpu.make_async_copy(xt_h.at[srco.at[b]], xlb.at[b], sems[b]).wait()
            pltpu.make_async_copy(rt_h.at[dsto.at[b]], xrb.at[b], sems[b]).wait()

        def compute(b):
            xl2d = xlb.at[b]
            xr2d = xrb.at[b]
            ob2d = ob.at[b]

            @pl.loop(0, CH, step=LANES)
            def _grp(g):
                for u in range(LANES):
                    e = g + u
                    xrow = xl2d.at[e]
                    rrow = xr2d.at[e]
                    xl0 = xrow[pl.ds(0, 16)]
                    xl1 = xrow[pl.ds(16, 16)]
                    t0 = xl0 + rrow[pl.ds(0, 16)]
                    t1 = xl1 + rrow[pl.ds(16, 16)]
                    t0 = jnp.maximum(t0, 0.2 * t0)
                    t1 = jnp.maximum(t1, 0.2 * t1)
                    a0 = jnp.sum(t0 * att0)
                    a1 = jnp.sum(t1 * att1)
                    p0 = jnp.exp(jnp.full((16,), a0, jnp.float32))
                    p1 = jnp.exp(jnp.full((16,), a1, jnp.float32))
                    orow = ob2d.at[e]
                    orow[pl.ds(0, 16)] = p0 * xl0
                    orow[pl.ds(16, 16)] = p1 * xl1
                    pb = jnp.where(lid == 0, p0, jnp.where(lid == 1, p1, 0.0))
                    plsc.store_scatter(orow, [lid + 32], pb, mask=lid < 8)

            pltpu.sync_copy(ob.at[b], accs.at[dsti.at[b]], add=True)

        fire(0, 0)

        @pl.loop(0, NCHUNK, step=2)
        def _pair(kk):
            drain(0)
            fire(1, kk + 1)
            compute(0)
            drain(1)
            fire(0, jnp.minimum(kk + 2, NCHUNK - 1))
            compute(1)

        drain(0)
        plsc.subcore_barrier()
        pltpu.sync_copy(accs.at[pl.ds(s * RPT, RPT)],
                        acc_out.at[c, pl.ds(s * RPT, RPT)])

    return k(xtab, rtab, att, soff, doff, dst, zat)


def _mk_tables(v):
    a = jnp.pad(v[:, :32], ((0, NP - N), (0, 0)))
    b = jnp.pad(v[:, 32:], ((0, NP - N), (0, 0)))
    return jnp.concatenate([a, b], axis=0)


def kernel(x, edge_index, hidden_state, enc_W1, enc_b1, enc_W2, enc_b2,
           g1_Wl, g1_bl, g1_Wr, g1_br, g1_att, g1_bias,
           g2_Wl, g2_bl, g2_Wr, g2_br, g2_att, g2_bias,
           gru_Wih, gru_bih, gru_Whh, gru_bhh,
           dec_W1, dec_b1, dec_W2, dec_b2):
    r1 = lambda v: v.reshape(1, -1)
    ar = jnp.arange(N, dtype=jnp.int32)
    padi = jnp.full((EP - E2,), N, jnp.int32)
    src = jnp.concatenate([edge_index[0], ar, padi])
    dst = jnp.concatenate([edge_index[1], ar, padi])
    soff = jnp.stack([src, src + NP])
    doff = jnp.stack([dst, dst + NP])
    zat = jnp.zeros((RPT, ACCW), jnp.float32)

    xl1, xr1 = _tc_encode(x, enc_W1, r1(enc_b1), enc_W2, r1(enc_b2),
                          g1_Wl, r1(g1_bl), g1_Wr, r1(g1_br))
    acc1 = _sc_gat(_mk_tables(xl1), _mk_tables(xr1), g1_att,
                   soff, doff, dst, zat)
    xl2, xr2 = _tc_mid(acc1[0, :N], acc1[1, :N], r1(g1_bias),
                       g2_Wl, r1(g2_bl), g2_Wr, r1(g2_br))
    acc2 = _sc_gat(_mk_tables(xl2), _mk_tables(xr2), g2_att,
                   soff, doff, dst, zat)
    out, new_hidden = _tc_final(acc2[0, :N], acc2[1, :N], r1(g2_bias),
                                hidden_state, gru_Wih, r1(gru_bih),
                                gru_Whh, r1(gru_bhh),
                                dec_W1, r1(dec_b1), dec_W2, r1(dec_b2))
    return (out, new_hidden)


# speedup vs baseline: 1.1959x; 1.1959x over previous
import dataclasses
import functools

import jax
import jax.numpy as jnp
from jax import lax
from jax.experimental import pallas as pl
from jax.experimental.pallas import tpu as pltpu
from jax.experimental.pallas import tpu_sc as plsc

N = 50000
E = 800000
HID = 64
HEADS = 4
DOUT = 7

NC = 2
NT = 16
LANES = 16

NP = 50016
RPT = NP // NT
CH = 48
E2 = E + N
NCHUNK = 1107
EPT = NCHUNK * CH
EP = EPT * NT
ACCW = 40

_ROWS = 2000
_GRID = N // _ROWS


def _tc_encode_body(x_ref, w1, b1, w2, b2, wl, bl, wr, br, xl_ref, xr_ref):
    h = jnp.maximum(jnp.dot(x_ref[...], w1[...],
                            preferred_element_type=jnp.float32) + b1[...], 0.0)
    h = jnp.dot(h, w2[...], preferred_element_type=jnp.float32) + b2[...]
    xl_ref[...] = jnp.dot(h, wl[...], preferred_element_type=jnp.float32) + bl[...]
    xr_ref[...] = jnp.dot(h, wr[...], preferred_element_type=jnp.float32) + br[...]


def _norm_heads(a, b):
    h0 = a[:, 0:16] / (a[:, 32:33] + 1e-16)
    h1 = a[:, 16:32] / (a[:, 33:34] + 1e-16)
    h2 = b[:, 0:16] / (b[:, 32:33] + 1e-16)
    h3 = b[:, 16:32] / (b[:, 33:34] + 1e-16)
    return jnp.concatenate([h0, h1, h2, h3], axis=1)


def _tc_mid_body(a_ref, b_ref, bias, wl, bl, wr, br, xl_ref, xr_ref):
    h = jnp.maximum(_norm_heads(a_ref[...], b_ref[...]) + bias[...], 0.0)
    xl_ref[...] = jnp.dot(h, wl[...], preferred_element_type=jnp.float32) + bl[...]
    xr_ref[...] = jnp.dot(h, wr[...], preferred_element_type=jnp.float32) + br[...]


def _tc_final_body(a_ref, b_ref, bias, hs_ref, wih, bih, whh, bhh,
                   dw1, db1, dw2, db2, out_ref, nh_ref):
    h = _norm_heads(a_ref[...], b_ref[...]) + bias[...]
    hs = hs_ref[...]
    gi = jnp.dot(h, wih[...], preferred_element_type=jnp.float32) + bih[...]
    gh = jnp.dot(hs, whh[...], preferred_element_type=jnp.float32) + bhh[...]
    r = jax.nn.sigmoid(gi[:, 0:64] + gh[:, 0:64])
    z = jax.nn.sigmoid(gi[:, 64:128] + gh[:, 64:128])
    n = jnp.tanh(gi[:, 128:192] + r * gh[:, 128:192])
    nh = (1.0 - z) * n + z * hs
    d = jnp.maximum(jnp.dot(nh, dw1[...], preferred_element_type=jnp.float32)
                    + db1[...], 0.0)
    out_ref[...] = jnp.dot(d, dw2[...], preferred_element_type=jnp.float32) + db2[...]
    nh_ref[...] = nh


def _full(shape):
    return pl.BlockSpec(shape, lambda i: tuple(0 for _ in shape))


def _rows(width):
    return pl.BlockSpec((_ROWS, width), lambda i: (i, 0))


def _tc_encode(x, w1, b1, w2, b2, wl, bl, wr, br):
    return pl.pallas_call(
        _tc_encode_body,
        grid=(_GRID,),
        in_specs=[_rows(8), _full((8, 64)), _full((1, 64)), _full((64, 64)),
                  _full((1, 64)), _full((64, 64)), _full((1, 64)),
                  _full((64, 64)), _full((1, 64))],
        out_specs=[_rows(64), _rows(64)],
        out_shape=[jax.ShapeDtypeStruct((N, 64), jnp.float32),
                   jax.ShapeDtypeStruct((N, 64), jnp.float32)],
    )(x, w1, b1, w2, b2, wl, bl, wr, br)


def _tc_mid(a, b, bias, wl, bl, wr, br):
    return pl.pallas_call(
        _tc_mid_body,
        grid=(_GRID,),
        in_specs=[_rows(ACCW), _rows(ACCW), _full((1, 64)), _full((64, 64)),
                  _full((1, 64)), _full((64, 64)), _full((1, 64))],
        out_specs=[_rows(64), _rows(64)],
        out_shape=[jax.ShapeDtypeStruct((N, 64), jnp.float32),
                   jax.ShapeDtypeStruct((N, 64), jnp.float32)],
    )(a, b, bias, wl, bl, wr, br)


def _tc_final(a, b, bias, hs, wih, bih, whh, bhh, dw1, db1, dw2, db2):
    return pl.pallas_call(
        _tc_final_body,
        grid=(_GRID,),
        in_specs=[_rows(ACCW), _rows(ACCW), _full((1, 64)), _rows(64),
                  _full((64, 192)), _full((1, 192)), _full((64, 192)),
                  _full((1, 192)), _full((64, 64)), _full((1, 64)),
                  _full((64, 7)), _full((1, 7))],
        out_specs=[_rows(7), _rows(64)],
        out_shape=[jax.ShapeDtypeStruct((N, 7), jnp.float32),
                   jax.ShapeDtypeStruct((N, 64), jnp.float32)],
    )(a, b, bias, hs, wih, bih, whh, bhh, dw1, db1, dw2, db2)


def _sc_gat(xtab, rtab, att, src, dst, zat):
    mesh = plsc.VectorSubcoreMesh(core_axis_name="c", subcore_axis_name="s")
    cp = pltpu.CompilerParams()
    for fld, val in (("needs_layout_passes", False),
                     ("use_tc_tiling_on_sc", False)):
        if fld in pltpu.CompilerParams.__dataclass_fields__:
            cp = dataclasses.replace(cp, **{fld: val})

    @functools.partial(
        pl.kernel,
        mesh=mesh,
        compiler_params=cp,
        out_type=jax.ShapeDtypeStruct((NC, NP, ACCW), jnp.float32),
        scratch_types=[
            pltpu.VMEM((CH,), jnp.int32),
            pltpu.VMEM((CH,), jnp.int32),
            pltpu.VMEM((CH,), jnp.int32),
            pltpu.VMEM((CH, 32), jnp.float32),
            pltpu.VMEM((CH, 32), jnp.float32),
            pltpu.VMEM((CH, ACCW), jnp.float32),
            pltpu.VMEM((2, 16), jnp.float32),
            pltpu.VMEM_SHARED((NP, ACCW), jnp.float32),
        ],
    )
    def k(xt_h, rt_h, att_h, src_h, dst_h, z_h, acc_out,
          dsti, srco, dsto, xlb, xrb, ob, attb, accs):
        c = lax.axis_index("c")
        s = lax.axis_index("s")
        pltpu.sync_copy(att_h.at[pl.ds(2 * c, 2)], attb)
        pltpu.sync_copy(z_h, accs.at[pl.ds(s * RPT, RPT)])
        plsc.subcore_barrier()
        off = c * NP
        lid = lax.iota(jnp.int32, 16)
        att0 = attb[0]
        att1 = attb[1]
        perms = [((lid + sh) & 15).reshape(16, 1) for sh in (8, 4, 2, 1)]
        gdn = lax.GatherDimensionNumbers(offset_dims=(),
                                         collapsed_slice_dims=(0,),
                                         start_index_map=(0,))

        def _allsum(v):
            for pm in perms:
                v = v + lax.gather(v, pm, gdn, slice_sizes=(1,),
                                   mode=lax.GatherScatterMode.PROMISE_IN_BOUNDS)
            return v

        @pl.loop(0, NCHUNK)
        def _chunk(kk):
            base = (s * NCHUNK + kk) * CH
            pltpu.sync_copy(src_h.at[pl.ds(base, CH)], srco)
            pltpu.sync_copy(dst_h.at[pl.ds(base, CH)], dsti)

            @pl.loop(0, CH, step=LANES)
            def _off(i):
                srco[pl.ds(i, LANES)] = srco[pl.ds(i, LANES)] + off
                dsto[pl.ds(i, LANES)] = dsti[pl.ds(i, LANES)] + off

            pltpu.sync_copy(xt_h.at[srco], xlb)
            pltpu.sync_copy(rt_h.at[dsto], xrb)

            @pl.loop(0, CH, step=LANES)
            def _grp(g):
                for u in range(LANES):
                    e = g + u
                    xrow = xlb.at[e]
                    rrow = xrb.at[e]
                    xl0 = xrow[pl.ds(0, 16)]
                    xl1 = xrow[pl.ds(16, 16)]
                    t0 = xl0 + rrow[pl.ds(0, 16)]
                    t1 = xl1 + rrow[pl.ds(16, 16)]
                    t0 = jnp.maximum(t0, 0.2 * t0)
                    t1 = jnp.maximum(t1, 0.2 * t1)
                    p0 = jnp.exp(_allsum(t0 * att0))
                    p1 = jnp.exp(_allsum(t1 * att1))
                    orow = ob.at[e]
                    orow[pl.ds(0, 16)] = p0 * xl0
                    orow[pl.ds(16, 16)] = p1 * xl1
                    pb = jnp.where(lid == 0, p0, jnp.where(lid == 1, p1, 0.0))
                    plsc.store_scatter(orow, [lid + 32], pb, mask=lid < 8)

            pltpu.sync_copy(ob, accs.at[dsti], add=True)

        plsc.subcore_barrier()
        pltpu.sync_copy(accs.at[pl.ds(s * RPT, RPT)],
                        acc_out.at[c, pl.ds(s * RPT, RPT)])

    return k(xtab, rtab, att, src, dst, zat)


def _mk_tables(v):
    a = jnp.pad(v[:, :32], ((0, NP - N), (0, 0)))
    b = jnp.pad(v[:, 32:], ((0, NP - N), (0, 0)))
    return jnp.concatenate([a, b], axis=0)


def kernel(x, edge_index, hidden_state, enc_W1, enc_b1, enc_W2, enc_b2,
           g1_Wl, g1_bl, g1_Wr, g1_br, g1_att, g1_bias,
           g2_Wl, g2_bl, g2_Wr, g2_br, g2_att, g2_bias,
           gru_Wih, gru_bih, gru_Whh, gru_bhh,
           dec_W1, dec_b1, dec_W2, dec_b2):
    r1 = lambda v: v.reshape(1, -1)
    ar = jnp.arange(N, dtype=jnp.int32)
    padi = jnp.full((EP - E2,), N, jnp.int32)
    src = jnp.concatenate([edge_index[0], ar, padi])
    dst = jnp.concatenate([edge_index[1], ar, padi])
    zat = jnp.zeros((RPT, ACCW), jnp.float32)

    xl1, xr1 = _tc_encode(x, enc_W1, r1(enc_b1), enc_W2, r1(enc_b2),
                          g1_Wl, r1(g1_bl), g1_Wr, r1(g1_br))
    acc1 = _sc_gat(_mk_tables(xl1), _mk_tables(xr1), g1_att, src, dst, zat)
    xl2, xr2 = _tc_mid(acc1[0, :N], acc1[1, :N], r1(g1_bias),
                       g2_Wl, r1(g2_bl), g2_Wr, r1(g2_br))
    acc2 = _sc_gat(_mk_tables(xl2), _mk_tables(xr2), g2_att, src, dst, zat)
    out, new_hidden = _tc_final(acc2[0, :N], acc2[1, :N], r1(g2_bias),
                                hidden_state, gru_Wih, r1(gru_bih),
                                gru_Whh, r1(gru_bhh),
                                dec_W1, r1(dec_b1), dec_W2, r1(dec_b2))
    return (out, new_hidden)


# speedup vs baseline: 1.4774x; 1.2353x over previous
import dataclasses
import functools

import jax
import jax.numpy as jnp
from jax import lax
from jax.experimental import pallas as pl
from jax.experimental.pallas import tpu as pltpu
from jax.experimental.pallas import tpu_sc as plsc

N = 50000
E = 800000
HID = 64
HEADS = 4
DOUT = 7

NC = 2
NT = 16
LANES = 16

NP = 50016
RPT = NP // NT
CH = 128
E2 = E + N
NCHUNK = 416
EPT = NCHUNK * CH
EP = EPT * NT
ACCW = 32

_ROWS = 2000
_GRID = N // _ROWS


def _tc_encode_body(x_ref, w1, b1, w2, b2, wl, bl, wr, br, xl_ref, xr_ref):
    h = jnp.maximum(jnp.dot(x_ref[...], w1[...],
                            preferred_element_type=jnp.float32) + b1[...], 0.0)
    h = jnp.dot(h, w2[...], preferred_element_type=jnp.float32) + b2[...]
    xl_ref[...] = jnp.dot(h, wl[...], preferred_element_type=jnp.float32) + bl[...]
    xr_ref[...] = jnp.dot(h, wr[...], preferred_element_type=jnp.float32) + br[...]


def _norm_heads(a, b, pa, pb):
    h0 = a[:, 0:16] / (pa[:, 0:1] + 1e-16)
    h1 = a[:, 16:32] / (pa[:, 1:2] + 1e-16)
    h2 = b[:, 0:16] / (pb[:, 0:1] + 1e-16)
    h3 = b[:, 16:32] / (pb[:, 1:2] + 1e-16)
    return jnp.concatenate([h0, h1, h2, h3], axis=1)


def _tc_mid_body(a_ref, b_ref, pa_ref, pb_ref, bias, wl, bl, wr, br,
                 xl_ref, xr_ref):
    h = jnp.maximum(_norm_heads(a_ref[...], b_ref[...], pa_ref[...],
                                pb_ref[...]) + bias[...], 0.0)
    xl_ref[...] = jnp.dot(h, wl[...], preferred_element_type=jnp.float32) + bl[...]
    xr_ref[...] = jnp.dot(h, wr[...], preferred_element_type=jnp.float32) + br[...]


def _tc_final_body(a_ref, b_ref, pa_ref, pb_ref, bias, hs_ref, wih, bih,
                   whh, bhh, dw1, db1, dw2, db2, out_ref, nh_ref):
    h = _norm_heads(a_ref[...], b_ref[...], pa_ref[...],
                    pb_ref[...]) + bias[...]
    hs = hs_ref[...]
    gi = jnp.dot(h, wih[...], preferred_element_type=jnp.float32) + bih[...]
    gh = jnp.dot(hs, whh[...], preferred_element_type=jnp.float32) + bhh[...]
    r = jax.nn.sigmoid(gi[:, 0:64] + gh[:, 0:64])
    z = jax.nn.sigmoid(gi[:, 64:128] + gh[:, 64:128])
    n = jnp.tanh(gi[:, 128:192] + r * gh[:, 128:192])
    nh = (1.0 - z) * n + z * hs
    d = jnp.maximum(jnp.dot(nh, dw1[...], preferred_element_type=jnp.float32)
                    + db1[...], 0.0)
    out_ref[...] = jnp.dot(d, dw2[...], preferred_element_type=jnp.float32) + db2[...]
    nh_ref[...] = nh


def _full(shape):
    return pl.BlockSpec(shape, lambda i: tuple(0 for _ in shape))


def _rows(width):
    return pl.BlockSpec((_ROWS, width), lambda i: (i, 0))


def _tc_encode(x, w1, b1, w2, b2, wl, bl, wr, br):
    return pl.pallas_call(
        _tc_encode_body,
        grid=(_GRID,),
        in_specs=[_rows(8), _full((8, 64)), _full((1, 64)), _full((64, 64)),
                  _full((1, 64)), _full((64, 64)), _full((1, 64)),
                  _full((64, 64)), _full((1, 64))],
        out_specs=[_rows(64), _rows(64)],
        out_shape=[jax.ShapeDtypeStruct((N, 64), jnp.float32),
                   jax.ShapeDtypeStruct((N, 64), jnp.float32)],
    )(x, w1, b1, w2, b2, wl, bl, wr, br)


def _tc_mid(a, b, pa, pb, bias, wl, bl, wr, br):
    return pl.pallas_call(
        _tc_mid_body,
        grid=(_GRID,),
        in_specs=[_rows(ACCW), _rows(ACCW), _rows(8), _rows(8),
                  _full((1, 64)), _full((64, 64)),
                  _full((1, 64)), _full((64, 64)), _full((1, 64))],
        out_specs=[_rows(64), _rows(64)],
        out_shape=[jax.ShapeDtypeStruct((N, 64), jnp.float32),
                   jax.ShapeDtypeStruct((N, 64), jnp.float32)],
    )(a, b, pa, pb, bias, wl, bl, wr, br)


def _tc_final(a, b, pa, pb, bias, hs, wih, bih, whh, bhh, dw1, db1, dw2, db2):
    return pl.pallas_call(
        _tc_final_body,
        grid=(_GRID,),
        in_specs=[_rows(ACCW), _rows(ACCW), _rows(8), _rows(8),
                  _full((1, 64)), _rows(64),
                  _full((64, 192)), _full((1, 192)), _full((64, 192)),
                  _full((1, 192)), _full((64, 64)), _full((1, 64)),
                  _full((64, 7)), _full((1, 7))],
        out_specs=[_rows(7), _rows(64)],
        out_shape=[jax.ShapeDtypeStruct((N, 7), jnp.float32),
                   jax.ShapeDtypeStruct((N, 64), jnp.float32)],
    )(a, b, pa, pb, bias, hs, wih, bih, whh, bhh, dw1, db1, dw2, db2)


def _sc_mesh_cp():
    mesh = plsc.VectorSubcoreMesh(core_axis_name="c", subcore_axis_name="s")
    cp = pltpu.CompilerParams()
    for fld, val in (("needs_layout_passes", False),
                     ("use_tc_tiling_on_sc", False)):
        if fld in pltpu.CompilerParams.__dataclass_fields__:
            cp = dataclasses.replace(cp, **{fld: val})
    return mesh, cp


def _sc_den(xtab, rtab, att, soff, doff, dst, zp):
    mesh, cp = _sc_mesh_cp()

    @functools.partial(
        pl.kernel,
        mesh=mesh,
        compiler_params=cp,
        out_type=(jax.ShapeDtypeStruct((NC, NP, 8), jnp.float32),
                  jax.ShapeDtypeStruct((NC, 2 * EP), jnp.float32)),
        scratch_types=[
            pltpu.VMEM((2, CH), jnp.int32),
            pltpu.VMEM((2, CH), jnp.int32),
            pltpu.VMEM((2, CH), jnp.int32),
            pltpu.VMEM((2, CH, 32), jnp.float32),
            pltpu.VMEM((2, CH, 32), jnp.float32),
            pltpu.VMEM((2, CH, 8), jnp.float32),
            pltpu.VMEM((2, 2 * CH), jnp.float32),
            pltpu.VMEM((2, 16), jnp.float32),
            pltpu.SemaphoreType.DMA,
            pltpu.SemaphoreType.DMA,
            pltpu.VMEM_SHARED((NP, 8), jnp.float32),
        ],
    )
    def k(xt_h, rt_h, att_h, soff_h, doff_h, dst_h, z_h, pacc_out, pv_out,
          dsti, srco, dsto, xlb, xrb, pb8, pvf, attb, sem0, sem1, accp):
        c = lax.axis_index("c")
        s = lax.axis_index("s")
        pltpu.sync_copy(att_h.at[pl.ds(2 * c, 2)], attb)
        pltpu.sync_copy(z_h, accp.at[pl.ds(s * RPT, RPT)])
        plsc.subcore_barrier()
        lid = lax.iota(jnp.int32, 16)
        att0 = attb[0]
        att1 = attb[1]
        sems = (sem0, sem1)
        cbase = s * NCHUNK
        perms = [((lid + sh) & 15).reshape(16, 1) for sh in (8, 4, 2, 1)]
        gdn = lax.GatherDimensionNumbers(offset_dims=(),
                                         collapsed_slice_dims=(0,),
                                         start_index_map=(0,))

        def _allsum(v):
            for pm in perms:
                v = v + lax.gather(v, pm, gdn, slice_sizes=(1,),
                                   mode=lax.GatherScatterMode.PROMISE_IN_BOUNDS)
            return v

        def fire(b, kk):
            base = (cbase + kk) * CH
            pltpu.sync_copy(soff_h.at[c, pl.ds(base, CH)], srco.at[b])
            pltpu.sync_copy(doff_h.at[c, pl.ds(base, CH)], dsto.at[b])
            pltpu.sync_copy(dst_h.at[pl.ds(base, CH)], dsti.at[b])
            pltpu.async_copy(xt_h.at[srco.at[b]], xlb.at[b], sems[b])
            pltpu.async_copy(rt_h.at[dsto.at[b]], xrb.at[b], sems[b])

        def drain(b):
            pltpu.make_async_copy(xt_h.at[srco.at[b]], xlb.at[b], sems[b]).wait()
            pltpu.make_async_copy(rt_h.at[dsto.at[b]], xrb.at[b], sems[b]).wait()

        def compute(b, kk):
            xl2d = xlb.at[b]
            xr2d = xrb.at[b]
            pb2d = pb8.at[b]
            pvfl = pvf.at[b]

            @pl.loop(0, CH, step=LANES)
            def _grp(g):
                for u in range(LANES):
                    e = g + u
                    xrow = xl2d.at[e]
                    rrow = xr2d.at[e]
                    t0 = xrow[pl.ds(0, 16)] + rrow[pl.ds(0, 16)]
                    t1 = xrow[pl.ds(16, 16)] + rrow[pl.ds(16, 16)]
                    t0 = jnp.maximum(t0, 0.2 * t0)
                    t1 = jnp.maximum(t1, 0.2 * t1)
                    p0 = jnp.exp(_allsum(t0 * att0))
                    p1 = jnp.exp(_allsum(t1 * att1))
                    pb = jnp.where(lid == 0, p0, jnp.where(lid == 1, p1, 0.0))
                    plsc.store_scatter(pb2d.at[e], [lid], pb, mask=lid < 8)
                    plsc.store_scatter(pvfl, [2 * e + lid], pb, mask=lid < 2)

            base = (cbase + kk) * CH
            pltpu.sync_copy(pvfl, pv_out.at[c, pl.ds(2 * base, 2 * CH)])
            pltpu.sync_copy(pb2d, accp.at[dsti.at[b]], add=True)

        fire(0, 0)

        @pl.loop(0, NCHUNK, step=2)
        def _pair(kk):
            drain(0)
            fire(1, kk + 1)
            compute(0, kk)
            drain(1)
            fire(0, jnp.minimum(kk + 2, NCHUNK - 1))
            compute(1, kk + 1)

        drain(0)
        plsc.subcore_barrier()
        pltpu.sync_copy(accp.at[pl.ds(s * RPT, RPT)],
                        pacc_out.at[c, pl.ds(s * RPT, RPT)])

    return k(xtab, rtab, att, soff, doff, dst, zp)


def _sc_agg(xtab, soff, dst, pvals, zat):
    mesh, cp = _sc_mesh_cp()

    @functools.partial(
        pl.kernel,
        mesh=mesh,
        compiler_params=cp,
        out_type=jax.ShapeDtypeStruct((NC, NP, ACCW), jnp.float32),
        scratch_types=[
            pltpu.VMEM((2, CH), jnp.int32),
            pltpu.VMEM((2, CH), jnp.int32),
            pltpu.VMEM((2, CH, 32), jnp.float32),
            pltpu.VMEM((2, 2 * CH), jnp.float32),
            pltpu.VMEM((2, CH, ACCW), jnp.float32),
            pltpu.SemaphoreType.DMA,
            pltpu.SemaphoreType.DMA,
            pltpu.VMEM_SHARED((NP, ACCW), jnp.float32),
        ],
    )
    def k(xt_h, soff_h, dst_h, pv_h, z_h, acc_out,
          dsti, srco, xlb, pvf, ob, sem0, sem1, accs):
        c = lax.axis_index("c")
        s = lax.axis_index("s")
        pltpu.sync_copy(z_h, accs.at[pl.ds(s * RPT, RPT)])
        plsc.subcore_barrier()
        sems = (sem0, sem1)
        cbase = s * NCHUNK

        def fire(b, kk):
            base = (cbase + kk) * CH
            pltpu.sync_copy(soff_h.at[c, pl.ds(base, CH)], srco.at[b])
            pltpu.sync_copy(dst_h.at[pl.ds(base, CH)], dsti.at[b])
            pltpu.sync_copy(pv_h.at[c, pl.ds(2 * base, 2 * CH)], pvf.at[b])
            pltpu.async_copy(xt_h.at[srco.at[b]], xlb.at[b], sems[b])

        def drain(b):
            pltpu.make_async_copy(xt_h.at[srco.at[b]], xlb.at[b], sems[b]).wait()

        def compute(b):
            xl2d = xlb.at[b]
            ob2d = ob.at[b]
            pvfl = pvf.at[b]

            @pl.loop(0, CH, step=8)
            def _grp(g):
                pvec = pvfl[pl.ds(2 * g, 16)]
                for u in range(8):
                    e = g + u
                    xrow = xl2d.at[e]
                    orow = ob2d.at[e]
                    p0 = jnp.full((16,), pvec[2 * u], jnp.float32)
                    p1 = jnp.full((16,), pvec[2 * u + 1], jnp.float32)
                    orow[pl.ds(0, 16)] = p0 * xrow[pl.ds(0, 16)]
                    orow[pl.ds(16, 16)] = p1 * xrow[pl.ds(16, 16)]

            pltpu.sync_copy(ob2d, accs.at[dsti.at[b]], add=True)

        fire(0, 0)

        @pl.loop(0, NCHUNK, step=2)
        def _pair(kk):
            drain(0)
            fire(1, kk + 1)
            compute(0)
            drain(1)
            fire(0, jnp.minimum(kk + 2, NCHUNK - 1))
            compute(1)

        drain(0)
        plsc.subcore_barrier()
        pltpu.sync_copy(accs.at[pl.ds(s * RPT, RPT)],
                        acc_out.at[c, pl.ds(s * RPT, RPT)])

    return k(xtab, soff, dst, pvals, zat)


def _mk_tables(v):
    a = jnp.pad(v[:, :32], ((0, NP - N), (0, 0)))
    b = jnp.pad(v[:, 32:], ((0, NP - N), (0, 0)))
    return jnp.concatenate([a, b], axis=0)


def kernel(x, edge_index, hidden_state, enc_W1, enc_b1, enc_W2, enc_b2,
           g1_Wl, g1_bl, g1_Wr, g1_br, g1_att, g1_bias,
           g2_Wl, g2_bl, g2_Wr, g2_br, g2_att, g2_bias,
           gru_Wih, gru_bih, gru_Whh, gru_bhh,
           dec_W1, dec_b1, dec_W2, dec_b2):
    r1 = lambda v: v.reshape(1, -1)
    ar = jnp.arange(N, dtype=jnp.int32)
    padi = jnp.full((EP - E2,), N, jnp.int32)
    src = jnp.concatenate([edge_index[0], ar, padi])
    dst = jnp.concatenate([edge_index[1], ar, padi])
    soff = jnp.stack([src, src + NP])
    doff = jnp.stack([dst, dst + NP])
    zat = jnp.zeros((RPT, ACCW), jnp.float32)
    zp = jnp.zeros((RPT, 8), jnp.float32)

    xl1, xr1 = _tc_encode(x, enc_W1, r1(enc_b1), enc_W2, r1(enc_b2),
                          g1_Wl, r1(g1_bl), g1_Wr, r1(g1_br))
    xt1 = _mk_tables(xl1)
    pac1, pv1 = _sc_den(xt1, _mk_tables(xr1), g1_att, soff, doff, dst, zp)
    acc1 = _sc_agg(xt1, soff, dst, pv1, zat)
    xl2, xr2 = _tc_mid(acc1[0, :N], acc1[1, :N],
                       pac1[0, :N], pac1[1, :N], r1(g1_bias),
                       g2_Wl, r1(g2_bl), g2_Wr, r1(g2_br))
    xt2 = _mk_tables(xl2)
    pac2, pv2 = _sc_den(xt2, _mk_tables(xr2), g2_att, soff, doff, dst, zp)
    acc2 = _sc_agg(xt2, soff, dst, pv2, zat)
    out, new_hidden = _tc_final(acc2[0, :N], acc2[1, :N],
                                pac2[0, :N], pac2[1, :N], r1(g2_bias),
                                hidden_state, gru_Wih, r1(gru_bih),
                                gru_Whh, r1(gru_bhh),
                                dec_W1, r1(dec_b1), dec_W2, r1(dec_b2))
    return (out, new_hidden)


# speedup vs baseline: 1.9664x; 1.3310x over previous
import dataclasses
import functools

import jax
import jax.numpy as jnp
from jax import lax
from jax.experimental import pallas as pl
from jax.experimental.pallas import tpu as pltpu
from jax.experimental.pallas import tpu_sc as plsc

N = 50000
E = 800000
HID = 64
HEADS = 4
DOUT = 7

NC = 2
NT = 16
LANES = 16

NP = 50016
RPT = NP // NT
CH = 128
E2 = E + N
NCHUNK = 416
EPT = NCHUNK * CH
EP = EPT * NT
ACCW = 32

_ROWS = 2000
_GRID = N // _ROWS


def _tc_encode_body(x_ref, w1, b1, w2, b2, wl, bl, wr, br, xl_ref, xr_ref):
    h = jnp.maximum(jnp.dot(x_ref[...], w1[...],
                            preferred_element_type=jnp.float32) + b1[...], 0.0)
    h = jnp.dot(h, w2[...], preferred_element_type=jnp.float32) + b2[...]
    xl_ref[...] = jnp.dot(h, wl[...], preferred_element_type=jnp.float32) + bl[...]
    xr_ref[...] = jnp.dot(h, wr[...], preferred_element_type=jnp.float32) + br[...]


def _norm_heads(a, b, pa, pb):
    h0 = a[:, 0:16] / (pa[:, 0:1] + 1e-16)
    h1 = a[:, 16:32] / (pa[:, 1:2] + 1e-16)
    h2 = b[:, 0:16] / (pb[:, 0:1] + 1e-16)
    h3 = b[:, 16:32] / (pb[:, 1:2] + 1e-16)
    return jnp.concatenate([h0, h1, h2, h3], axis=1)


def _tc_mid_body(a_ref, b_ref, pa_ref, pb_ref, bias, wl, bl, wr, br,
                 xl_ref, xr_ref):
    h = jnp.maximum(_norm_heads(a_ref[...], b_ref[...], pa_ref[...],
                                pb_ref[...]) + bias[...], 0.0)
    xl_ref[...] = jnp.dot(h, wl[...], preferred_element_type=jnp.float32) + bl[...]
    xr_ref[...] = jnp.dot(h, wr[...], preferred_element_type=jnp.float32) + br[...]


def _tc_final_body(a_ref, b_ref, pa_ref, pb_ref, bias, hs_ref, wih, bih,
                   whh, bhh, dw1, db1, dw2, db2, out_ref, nh_ref):
    h = _norm_heads(a_ref[...], b_ref[...], pa_ref[...],
                    pb_ref[...]) + bias[...]
    hs = hs_ref[...]
    gi = jnp.dot(h, wih[...], preferred_element_type=jnp.float32) + bih[...]
    gh = jnp.dot(hs, whh[...], preferred_element_type=jnp.float32) + bhh[...]
    r = jax.nn.sigmoid(gi[:, 0:64] + gh[:, 0:64])
    z = jax.nn.sigmoid(gi[:, 64:128] + gh[:, 64:128])
    n = jnp.tanh(gi[:, 128:192] + r * gh[:, 128:192])
    nh = (1.0 - z) * n + z * hs
    d = jnp.maximum(jnp.dot(nh, dw1[...], preferred_element_type=jnp.float32)
                    + db1[...], 0.0)
    out_ref[...] = jnp.dot(d, dw2[...], preferred_element_type=jnp.float32) + db2[...]
    nh_ref[...] = nh


def _full(shape):
    return pl.BlockSpec(shape, lambda i: tuple(0 for _ in shape))


def _rows(width):
    return pl.BlockSpec((_ROWS, width), lambda i: (i, 0))


def _tc_encode(x, w1, b1, w2, b2, wl, bl, wr, br):
    return pl.pallas_call(
        _tc_encode_body,
        grid=(_GRID,),
        in_specs=[_rows(8), _full((8, 64)), _full((1, 64)), _full((64, 64)),
                  _full((1, 64)), _full((64, 64)), _full((1, 64)),
                  _full((64, 64)), _full((1, 64))],
        out_specs=[_rows(64), _rows(64)],
        out_shape=[jax.ShapeDtypeStruct((N, 64), jnp.float32),
                   jax.ShapeDtypeStruct((N, 64), jnp.float32)],
    )(x, w1, b1, w2, b2, wl, bl, wr, br)


def _tc_mid(a, b, pa, pb, bias, wl, bl, wr, br):
    return pl.pallas_call(
        _tc_mid_body,
        grid=(_GRID,),
        in_specs=[_rows(ACCW), _rows(ACCW), _rows(8), _rows(8),
                  _full((1, 64)), _full((64, 64)),
                  _full((1, 64)), _full((64, 64)), _full((1, 64))],
        out_specs=[_rows(64), _rows(64)],
        out_shape=[jax.ShapeDtypeStruct((N, 64), jnp.float32),
                   jax.ShapeDtypeStruct((N, 64), jnp.float32)],
    )(a, b, pa, pb, bias, wl, bl, wr, br)


def _tc_final(a, b, pa, pb, bias, hs, wih, bih, whh, bhh, dw1, db1, dw2, db2):
    return pl.pallas_call(
        _tc_final_body,
        grid=(_GRID,),
        in_specs=[_rows(ACCW), _rows(ACCW), _rows(8), _rows(8),
                  _full((1, 64)), _rows(64),
                  _full((64, 192)), _full((1, 192)), _full((64, 192)),
                  _full((1, 192)), _full((64, 64)), _full((1, 64)),
                  _full((64, 7)), _full((1, 7))],
        out_specs=[_rows(7), _rows(64)],
        out_shape=[jax.ShapeDtypeStruct((N, 7), jnp.float32),
                   jax.ShapeDtypeStruct((N, 64), jnp.float32)],
    )(a, b, pa, pb, bias, hs, wih, bih, whh, bhh, dw1, db1, dw2, db2)


def _sc_mesh_cp():
    mesh = plsc.VectorSubcoreMesh(core_axis_name="c", subcore_axis_name="s")
    cp = pltpu.CompilerParams()
    for fld, val in (("needs_layout_passes", False),
                     ("use_tc_tiling_on_sc", False)):
        if fld in pltpu.CompilerParams.__dataclass_fields__:
            cp = dataclasses.replace(cp, **{fld: val})
    return mesh, cp


def _sc_den(xtab, rtab, att, soff, doff, dst, zp):
    mesh, cp = _sc_mesh_cp()

    @functools.partial(
        pl.kernel,
        mesh=mesh,
        compiler_params=cp,
        out_type=(jax.ShapeDtypeStruct((NC, NP, 8), jnp.float32),
                  jax.ShapeDtypeStruct((NC, 2 * EP), jnp.float32)),
        scratch_types=[
            pltpu.VMEM((2, CH), jnp.int32),
            pltpu.VMEM((2, CH), jnp.int32),
            pltpu.VMEM((2, CH), jnp.int32),
            pltpu.VMEM((2, CH, 32), jnp.float32),
            pltpu.VMEM((2, CH, 32), jnp.float32),
            pltpu.VMEM((2, CH, 8), jnp.float32),
            pltpu.VMEM((2, 2 * CH), jnp.float32),
            pltpu.VMEM((2, 16), jnp.float32),
            pltpu.SemaphoreType.DMA,
            pltpu.SemaphoreType.DMA,
            pltpu.SemaphoreType.DMA,
            pltpu.SemaphoreType.DMA,
            pltpu.VMEM_SHARED((NP, 8), jnp.float32),
        ],
    )
    def k(xt_h, rt_h, att_h, soff_h, doff_h, dst_h, z_h, pacc_out, pv_out,
          dsti, srco, dsto, xlb, xrb, pb8, pvf, attb, sem0, sem1,
          semi0, semi1, accp):
        c = lax.axis_index("c")
        s = lax.axis_index("s")
        pltpu.sync_copy(att_h.at[pl.ds(2 * c, 2)], attb)
        pltpu.sync_copy(z_h, accp.at[pl.ds(s * RPT, RPT)])
        plsc.subcore_barrier()
        lid = lax.iota(jnp.int32, 16)
        att0 = attb[0]
        att1 = attb[1]
        sems = (sem0, sem1)
        cbase = s * NCHUNK
        perms = [((lid + sh) & 15).reshape(16, 1) for sh in (8, 4, 2, 1)]
        gdn = lax.GatherDimensionNumbers(offset_dims=(),
                                         collapsed_slice_dims=(0,),
                                         start_index_map=(0,))

        def _allsum(v):
            for pm in perms:
                v = v + lax.gather(v, pm, gdn, slice_sizes=(1,),
                                   mode=lax.GatherScatterMode.PROMISE_IN_BOUNDS)
            return v

        semi = (semi0, semi1)

        def fire_idx(b, kk):
            base = (cbase + kk) * CH
            pltpu.async_copy(soff_h.at[c, pl.ds(base, CH)], srco.at[b], semi[b])
            pltpu.async_copy(doff_h.at[c, pl.ds(base, CH)], dsto.at[b], semi[b])
            pltpu.async_copy(dst_h.at[pl.ds(base, CH)], dsti.at[b], semi[b])

        def wait_idx(b, kk):
            base = (cbase + kk) * CH
            pltpu.make_async_copy(soff_h.at[c, pl.ds(base, CH)], srco.at[b],
                                  semi[b]).wait()
            pltpu.make_async_copy(doff_h.at[c, pl.ds(base, CH)], dsto.at[b],
                                  semi[b]).wait()
            pltpu.make_async_copy(dst_h.at[pl.ds(base, CH)], dsti.at[b],
                                  semi[b]).wait()

        def fire(b, kk):
            wait_idx(b, kk)
            pltpu.async_copy(xt_h.at[srco.at[b]], xlb.at[b], sems[b])
            pltpu.async_copy(rt_h.at[dsto.at[b]], xrb.at[b], sems[b])

        def drain(b):
            pltpu.make_async_copy(xt_h.at[srco.at[b]], xlb.at[b], sems[b]).wait()
            pltpu.make_async_copy(rt_h.at[dsto.at[b]], xrb.at[b], sems[b]).wait()

        def compute(b, kk):
            xl2d = xlb.at[b]
            xr2d = xrb.at[b]
            pb2d = pb8.at[b]
            pvfl = pvf.at[b]

            @pl.loop(0, CH, step=LANES)
            def _grp(g):
                for u in range(LANES):
                    e = g + u
                    xrow = xl2d.at[e]
                    rrow = xr2d.at[e]
                    t0 = xrow[pl.ds(0, 16)] + rrow[pl.ds(0, 16)]
                    t1 = xrow[pl.ds(16, 16)] + rrow[pl.ds(16, 16)]
                    t0 = jnp.maximum(t0, 0.2 * t0)
                    t1 = jnp.maximum(t1, 0.2 * t1)
                    p0 = jnp.exp(_allsum(t0 * att0))
                    p1 = jnp.exp(_allsum(t1 * att1))
                    pb = jnp.where(lid == 0, p0, jnp.where(lid == 1, p1, 0.0))
                    plsc.store_scatter(pb2d.at[e], [lid], pb, mask=lid < 8)
                    plsc.store_scatter(pvfl, [2 * e + lid], pb, mask=lid < 2)

            base = (cbase + kk) * CH
            pltpu.sync_copy(pvfl, pv_out.at[c, pl.ds(2 * base, 2 * CH)])
            pltpu.sync_copy(pb2d, accp.at[dsti.at[b]], add=True)

        fire_idx(0, 0)
        fire(0, 0)
        fire_idx(1, 1)

        @pl.loop(0, NCHUNK, step=2)
        def _pair(kk):
            drain(0)
            fire(1, kk + 1)
            compute(0, kk)
            fire_idx(0, jnp.minimum(kk + 2, NCHUNK - 1))
            drain(1)
            fire(0, jnp.minimum(kk + 2, NCHUNK - 1))
            compute(1, kk + 1)
            fire_idx(1, jnp.minimum(kk + 3, NCHUNK - 1))

        drain(0)
        wait_idx(1, NCHUNK - 1)
        plsc.subcore_barrier()
        pltpu.sync_copy(accp.at[pl.ds(s * RPT, RPT)],
                        pacc_out.at[c, pl.ds(s * RPT, RPT)])

    return k(xtab, rtab, att, soff, doff, dst, zp)


def _sc_agg(xtab, soff, dst, pvals, zat):
    mesh, cp = _sc_mesh_cp()

    @functools.partial(
        pl.kernel,
        mesh=mesh,
        compiler_params=cp,
        out_type=jax.ShapeDtypeStruct((NC, NP, ACCW), jnp.float32),
        scratch_types=[
            pltpu.VMEM((2, CH), jnp.int32),
            pltpu.VMEM((2, CH), jnp.int32),
            pltpu.VMEM((2, CH, 32), jnp.float32),
            pltpu.VMEM((2, 2 * CH), jnp.float32),
            pltpu.VMEM((2, CH, ACCW), jnp.float32),
            pltpu.SemaphoreType.DMA,
            pltpu.SemaphoreType.DMA,
            pltpu.SemaphoreType.DMA,
            pltpu.SemaphoreType.DMA,
            pltpu.VMEM_SHARED((NP, ACCW), jnp.float32),
        ],
    )
    def k(xt_h, soff_h, dst_h, pv_h, z_h, acc_out,
          dsti, srco, xlb, pvf, ob, sem0, sem1, semi0, semi1, accs):
        c = lax.axis_index("c")
        s = lax.axis_index("s")
        pltpu.sync_copy(z_h, accs.at[pl.ds(s * RPT, RPT)])
        plsc.subcore_barrier()
        sems = (sem0, sem1)
        semi = (semi0, semi1)
        cbase = s * NCHUNK

        def fire_idx(b, kk):
            base = (cbase + kk) * CH
            pltpu.async_copy(soff_h.at[c, pl.ds(base, CH)], srco.at[b], semi[b])
            pltpu.async_copy(dst_h.at[pl.ds(base, CH)], dsti.at[b], semi[b])
            pltpu.async_copy(pv_h.at[c, pl.ds(2 * base, 2 * CH)], pvf.at[b],
                             semi[b])

        def wait_idx(b, kk):
            base = (cbase + kk) * CH
            pltpu.make_async_copy(soff_h.at[c, pl.ds(base, CH)], srco.at[b],
                                  semi[b]).wait()
            pltpu.make_async_copy(dst_h.at[pl.ds(base, CH)], dsti.at[b],
                                  semi[b]).wait()
            pltpu.make_async_copy(pv_h.at[c, pl.ds(2 * base, 2 * CH)],
                                  pvf.at[b], semi[b]).wait()

        def fire(b, kk):
            wait_idx(b, kk)
            pltpu.async_copy(xt_h.at[srco.at[b]], xlb.at[b], sems[b])

        def drain(b):
            pltpu.make_async_copy(xt_h.at[srco.at[b]], xlb.at[b], sems[b]).wait()

        def compute(b):
            xl2d = xlb.at[b]
            ob2d = ob.at[b]
            pvfl = pvf.at[b]

            @pl.loop(0, CH, step=8)
            def _grp(g):
                pvec = pvfl[pl.ds(2 * g, 16)]
                for u in range(8):
                    e = g + u
                    xrow = xl2d.at[e]
                    orow = ob2d.at[e]
                    p0 = jnp.full((16,), pvec[2 * u], jnp.float32)
                    p1 = jnp.full((16,), pvec[2 * u + 1], jnp.float32)
                    orow[pl.ds(0, 16)] = p0 * xrow[pl.ds(0, 16)]
                    orow[pl.ds(16, 16)] = p1 * xrow[pl.ds(16, 16)]

            pltpu.sync_copy(ob2d, accs.at[dsti.at[b]], add=True)

        fire_idx(0, 0)
        fire(0, 0)
        fire_idx(1, 1)

        @pl.loop(0, NCHUNK, step=2)
        def _pair(kk):
            drain(0)
            fire(1, kk + 1)
            compute(0)
            fire_idx(0, jnp.minimum(kk + 2, NCHUNK - 1))
            drain(1)
            fire(0, jnp.minimum(kk + 2, NCHUNK - 1))
            compute(1)
            fire_idx(1, jnp.minimum(kk + 3, NCHUNK - 1))

        drain(0)
        wait_idx(1, NCHUNK - 1)
        plsc.subcore_barrier()
        pltpu.sync_copy(accs.at[pl.ds(s * RPT, RPT)],
                        acc_out.at[c, pl.ds(s * RPT, RPT)])

    return k(xtab, soff, dst, pvals, zat)


def _mk_tables(v):
    a = jnp.pad(v[:, :32], ((0, NP - N), (0, 0)))
    b = jnp.pad(v[:, 32:], ((0, NP - N), (0, 0)))
    return jnp.concatenate([a, b], axis=0)


def kernel(x, edge_index, hidden_state, enc_W1, enc_b1, enc_W2, enc_b2,
           g1_Wl, g1_bl, g1_Wr, g1_br, g1_att, g1_bias,
           g2_Wl, g2_bl, g2_Wr, g2_br, g2_att, g2_bias,
           gru_Wih, gru_bih, gru_Whh, gru_bhh,
           dec_W1, dec_b1, dec_W2, dec_b2):
    r1 = lambda v: v.reshape(1, -1)
    ar = jnp.arange(N, dtype=jnp.int32)
    padi = jnp.full((EP - E2,), N, jnp.int32)
    src = jnp.concatenate([edge_index[0], ar, padi])
    dst = jnp.concatenate([edge_index[1], ar, padi])
    soff = jnp.stack([src, src + NP])
    doff = jnp.stack([dst, dst + NP])
    zat = jnp.zeros((RPT, ACCW), jnp.float32)
    zp = jnp.zeros((RPT, 8), jnp.float32)

    xl1, xr1 = _tc_encode(x, enc_W1, r1(enc_b1), enc_W2, r1(enc_b2),
                          g1_Wl, r1(g1_bl), g1_Wr, r1(g1_br))
    xt1 = _mk_tables(xl1)
    pac1, pv1 = _sc_den(xt1, _mk_tables(xr1), g1_att, soff, doff, dst, zp)
    acc1 = _sc_agg(xt1, soff, dst, pv1, zat)
    xl2, xr2 = _tc_mid(acc1[0, :N], acc1[1, :N],
                       pac1[0, :N], pac1[1, :N], r1(g1_bias),
                       g2_Wl, r1(g2_bl), g2_Wr, r1(g2_br))
    xt2 = _mk_tables(xl2)
    pac2, pv2 = _sc_den(xt2, _mk_tables(xr2), g2_att, soff, doff, dst, zp)
    acc2 = _sc_agg(xt2, soff, dst, pv2, zat)
    out, new_hidden = _tc_final(acc2[0, :N], acc2[1, :N],
                                pac2[0, :N], pac2[1, :N], r1(g2_bias),
                                hidden_state, gru_Wih, r1(gru_bih),
                                gru_Whh, r1(gru_bhh),
                                dec_W1, r1(dec_b1), dec_W2, r1(dec_b2))
    return (out, new_hidden)


# speedup vs baseline: 2.0968x; 1.0663x over previous
import dataclasses
import functools

import jax
import jax.numpy as jnp
from jax import lax
from jax.experimental import pallas as pl
from jax.experimental.pallas import tpu as pltpu
from jax.experimental.pallas import tpu_sc as plsc

N = 50000
E = 800000
HID = 64
HEADS = 4
DOUT = 7

NC = 2
NT = 16
LANES = 16

NP = 50016
RPT = NP // NT
CH = 128
E2 = E + N
NCHUNK = 416
EPT = NCHUNK * CH
EP = EPT * NT
ACCW = 32

_ROWS = 2000
_GRID = N // _ROWS


def _tc_encode_body(x_ref, w1, b1, w2, b2, wl, bl, wr, br, xl_ref, xr_ref):
    h = jnp.maximum(jnp.dot(x_ref[...], w1[...],
                            preferred_element_type=jnp.float32) + b1[...], 0.0)
    h = jnp.dot(h, w2[...], preferred_element_type=jnp.float32) + b2[...]
    xl_ref[...] = jnp.dot(h, wl[...], preferred_element_type=jnp.float32) + bl[...]
    xr_ref[...] = jnp.dot(h, wr[...], preferred_element_type=jnp.float32) + br[...]


def _norm_heads(a, b, pa, pb):
    h0 = a[:, 0:16] / (pa[:, 0:1] + 1e-16)
    h1 = a[:, 16:32] / (pa[:, 1:2] + 1e-16)
    h2 = b[:, 0:16] / (pb[:, 0:1] + 1e-16)
    h3 = b[:, 16:32] / (pb[:, 1:2] + 1e-16)
    return jnp.concatenate([h0, h1, h2, h3], axis=1)


def _tc_mid_body(a_ref, b_ref, pa_ref, pb_ref, bias, wl, bl, wr, br,
                 xl_ref, xr_ref):
    h = jnp.maximum(_norm_heads(a_ref[...], b_ref[...], pa_ref[...],
                                pb_ref[...]) + bias[...], 0.0)
    xl_ref[...] = jnp.dot(h, wl[...], preferred_element_type=jnp.float32) + bl[...]
    xr_ref[...] = jnp.dot(h, wr[...], preferred_element_type=jnp.float32) + br[...]


def _tc_final_body(a_ref, b_ref, pa_ref, pb_ref, bias, hs_ref, wih, bih,
                   whh, bhh, dw1, db1, dw2, db2, out_ref, nh_ref):
    h = _norm_heads(a_ref[...], b_ref[...], pa_ref[...],
                    pb_ref[...]) + bias[...]
    hs = hs_ref[...]
    gi = jnp.dot(h, wih[...], preferred_element_type=jnp.float32) + bih[...]
    gh = jnp.dot(hs, whh[...], preferred_element_type=jnp.float32) + bhh[...]
    r = jax.nn.sigmoid(gi[:, 0:64] + gh[:, 0:64])
    z = jax.nn.sigmoid(gi[:, 64:128] + gh[:, 64:128])
    n = jnp.tanh(gi[:, 128:192] + r * gh[:, 128:192])
    nh = (1.0 - z) * n + z * hs
    d = jnp.maximum(jnp.dot(nh, dw1[...], preferred_element_type=jnp.float32)
                    + db1[...], 0.0)
    out_ref[...] = jnp.dot(d, dw2[...], preferred_element_type=jnp.float32) + db2[...]
    nh_ref[...] = nh


def _full(shape):
    return pl.BlockSpec(shape, lambda i: tuple(0 for _ in shape))


def _rows(width):
    return pl.BlockSpec((_ROWS, width), lambda i: (i, 0))


def _tc_encode(x, w1, b1, w2, b2, wl, bl, wr, br):
    return pl.pallas_call(
        _tc_encode_body,
        grid=(_GRID,),
        in_specs=[_rows(8), _full((8, 64)), _full((1, 64)), _full((64, 64)),
                  _full((1, 64)), _full((64, 64)), _full((1, 64)),
                  _full((64, 64)), _full((1, 64))],
        out_specs=[_rows(64), _rows(64)],
        out_shape=[jax.ShapeDtypeStruct((N, 64), jnp.float32),
                   jax.ShapeDtypeStruct((N, 64), jnp.float32)],
    )(x, w1, b1, w2, b2, wl, bl, wr, br)


def _tc_mid(a, b, pa, pb, bias, wl, bl, wr, br):
    return pl.pallas_call(
        _tc_mid_body,
        grid=(_GRID,),
        in_specs=[_rows(ACCW), _rows(ACCW), _rows(8), _rows(8),
                  _full((1, 64)), _full((64, 64)),
                  _full((1, 64)), _full((64, 64)), _full((1, 64))],
        out_specs=[_rows(64), _rows(64)],
        out_shape=[jax.ShapeDtypeStruct((N, 64), jnp.float32),
                   jax.ShapeDtypeStruct((N, 64), jnp.float32)],
    )(a, b, pa, pb, bias, wl, bl, wr, br)


def _tc_final(a, b, pa, pb, bias, hs, wih, bih, whh, bhh, dw1, db1, dw2, db2):
    return pl.pallas_call(
        _tc_final_body,
        grid=(_GRID,),
        in_specs=[_rows(ACCW), _rows(ACCW), _rows(8), _rows(8),
                  _full((1, 64)), _rows(64),
                  _full((64, 192)), _full((1, 192)), _full((64, 192)),
                  _full((1, 192)), _full((64, 64)), _full((1, 64)),
                  _full((64, 7)), _full((1, 7))],
        out_specs=[_rows(7), _rows(64)],
        out_shape=[jax.ShapeDtypeStruct((N, 7), jnp.float32),
                   jax.ShapeDtypeStruct((N, 64), jnp.float32)],
    )(a, b, pa, pb, bias, hs, wih, bih, whh, bhh, dw1, db1, dw2, db2)


def _sc_mesh_cp():
    mesh = plsc.VectorSubcoreMesh(core_axis_name="c", subcore_axis_name="s")
    cp = pltpu.CompilerParams()
    for fld, val in (("needs_layout_passes", False),
                     ("use_tc_tiling_on_sc", False)):
        if fld in pltpu.CompilerParams.__dataclass_fields__:
            cp = dataclasses.replace(cp, **{fld: val})
    return mesh, cp


def _sc_den(xtab, rtab, att, soff, doff, dst, zp):
    mesh, cp = _sc_mesh_cp()

    @functools.partial(
        pl.kernel,
        mesh=mesh,
        compiler_params=cp,
        out_type=(jax.ShapeDtypeStruct((NC, NP, 8), jnp.float32),
                  jax.ShapeDtypeStruct((NC, 2 * EP), jnp.float32)),
        scratch_types=[
            pltpu.VMEM((4, CH), jnp.int32),
            pltpu.VMEM((4, CH), jnp.int32),
            pltpu.VMEM((4, CH), jnp.int32),
            pltpu.VMEM((4, CH, 32), jnp.float32),
            pltpu.VMEM((4, CH, 32), jnp.float32),
            pltpu.VMEM((4, CH, 8), jnp.float32),
            pltpu.VMEM((4, 2 * CH), jnp.float32),
            pltpu.VMEM((2, 16), jnp.float32),
            pltpu.SemaphoreType.DMA,
            pltpu.SemaphoreType.DMA,
            pltpu.SemaphoreType.DMA,
            pltpu.SemaphoreType.DMA,
            pltpu.SemaphoreType.DMA,
            pltpu.SemaphoreType.DMA,
            pltpu.SemaphoreType.DMA,
            pltpu.SemaphoreType.DMA,
            pltpu.VMEM_SHARED((NP, 8), jnp.float32),
        ],
    )
    def k(xt_h, rt_h, att_h, soff_h, doff_h, dst_h, z_h, pacc_out, pv_out,
          dsti, srco, dsto, xlb, xrb, pb8, pvf, attb, sem0, sem1, sem2, sem3,
          semi0, semi1, semi2, semi3, accp):
        c = lax.axis_index("c")
        s = lax.axis_index("s")
        pltpu.sync_copy(att_h.at[pl.ds(2 * c, 2)], attb)
        pltpu.sync_copy(z_h, accp.at[pl.ds(s * RPT, RPT)])
        plsc.subcore_barrier()
        lid = lax.iota(jnp.int32, 16)
        att0 = attb[0]
        att1 = attb[1]
        sems = (sem0, sem1, sem2, sem3)
        cbase = s * NCHUNK
        perms = [((lid + sh) & 15).reshape(16, 1) for sh in (8, 4, 2, 1)]
        gdn = lax.GatherDimensionNumbers(offset_dims=(),
                                         collapsed_slice_dims=(0,),
                                         start_index_map=(0,))

        def _allsum(v):
            for pm in perms:
                v = v + lax.gather(v, pm, gdn, slice_sizes=(1,),
                                   mode=lax.GatherScatterMode.PROMISE_IN_BOUNDS)
            return v

        semi = (semi0, semi1, semi2, semi3)

        def fire_idx(b, kk):
            base = (cbase + kk) * CH
            pltpu.async_copy(soff_h.at[c, pl.ds(base, CH)], srco.at[b], semi[b])
            pltpu.async_copy(doff_h.at[c, pl.ds(base, CH)], dsto.at[b], semi[b])
            pltpu.async_copy(dst_h.at[pl.ds(base, CH)], dsti.at[b], semi[b])

        def wait_idx(b, kk):
            base = (cbase + kk) * CH
            pltpu.make_async_copy(soff_h.at[c, pl.ds(base, CH)], srco.at[b],
                                  semi[b]).wait()
            pltpu.make_async_copy(doff_h.at[c, pl.ds(base, CH)], dsto.at[b],
                                  semi[b]).wait()
            pltpu.make_async_copy(dst_h.at[pl.ds(base, CH)], dsti.at[b],
                                  semi[b]).wait()

        def fire(b, kk):
            wait_idx(b, kk)
            pltpu.async_copy(xt_h.at[srco.at[b]], xlb.at[b], sems[b])
            pltpu.async_copy(rt_h.at[dsto.at[b]], xrb.at[b], sems[b])

        def drain(b):
            pltpu.make_async_copy(xt_h.at[srco.at[b]], xlb.at[b], sems[b]).wait()
            pltpu.make_async_copy(rt_h.at[dsto.at[b]], xrb.at[b], sems[b]).wait()

        def compute(b, kk):
            xl2d = xlb.at[b]
            xr2d = xrb.at[b]
            pb2d = pb8.at[b]
            pvfl = pvf.at[b]

            @pl.loop(0, CH, step=LANES)
            def _grp(g):
                for u in range(LANES):
                    e = g + u
                    xrow = xl2d.at[e]
                    rrow = xr2d.at[e]
                    t0 = xrow[pl.ds(0, 16)] + rrow[pl.ds(0, 16)]
                    t1 = xrow[pl.ds(16, 16)] + rrow[pl.ds(16, 16)]
                    t0 = jnp.maximum(t0, 0.2 * t0)
                    t1 = jnp.maximum(t1, 0.2 * t1)
                    p0 = jnp.exp(_allsum(t0 * att0))
                    p1 = jnp.exp(_allsum(t1 * att1))
                    pb = jnp.where(lid == 0, p0, jnp.where(lid == 1, p1, 0.0))
                    plsc.store_scatter(pb2d.at[e], [lid], pb, mask=lid < 8)
                    plsc.store_scatter(pvfl, [2 * e + lid], pb, mask=lid < 2)

            base = (cbase + kk) * CH
            pltpu.sync_copy(pvfl, pv_out.at[c, pl.ds(2 * base, 2 * CH)])
            pltpu.sync_copy(pb2d, accp.at[dsti.at[b]], add=True)

        fire_idx(0, 0)
        fire_idx(1, 1)
        fire_idx(2, 2)
        fire(0, 0)
        fire(1, 1)
        fire(2, 2)
        fire_idx(3, 3)

        @pl.loop(0, NCHUNK, step=4)
        def _quad(kk):
            for d in range(4):
                drain(d)
                compute(d, kk + d)
                fire_idx(d, jnp.minimum(kk + d + 4, NCHUNK - 1))
                fire((d + 3) % 4, jnp.minimum(kk + d + 3, NCHUNK - 1))

        drain(0)
        drain(1)
        drain(2)
        wait_idx(3, NCHUNK - 1)
        plsc.subcore_barrier()
        pltpu.sync_copy(accp.at[pl.ds(s * RPT, RPT)],
                        pacc_out.at[c, pl.ds(s * RPT, RPT)])

    return k(xtab, rtab, att, soff, doff, dst, zp)


def _sc_agg(xtab, soff, dst, pvals, zat):
    mesh, cp = _sc_mesh_cp()

    @functools.partial(
        pl.kernel,
        mesh=mesh,
        compiler_params=cp,
        out_type=jax.ShapeDtypeStruct((NC, NP, ACCW), jnp.float32),
        scratch_types=[
            pltpu.VMEM((2, CH), jnp.int32),
            pltpu.VMEM((2, CH), jnp.int32),
            pltpu.VMEM((2, CH, 32), jnp.float32),
            pltpu.VMEM((2, 2 * CH), jnp.float32),
            pltpu.VMEM((2, CH, ACCW), jnp.float32),
            pltpu.SemaphoreType.DMA,
            pltpu.SemaphoreType.DMA,
            pltpu.SemaphoreType.DMA,
            pltpu.SemaphoreType.DMA,
            pltpu.VMEM_SHARED((NP, ACCW), jnp.float32),
        ],
    )
    def k(xt_h, soff_h, dst_h, pv_h, z_h, acc_out,
          dsti, srco, xlb, pvf, ob, sem0, sem1, semi0, semi1, accs):
        c = lax.axis_index("c")
        s = lax.axis_index("s")
        pltpu.sync_copy(z_h, accs.at[pl.ds(s * RPT, RPT)])
        plsc.subcore_barrier()
        sems = (sem0, sem1)
        semi = (semi0, semi1)
        cbase = s * NCHUNK

        def fire_idx(b, kk):
            base = (cbase + kk) * CH
            pltpu.async_copy(soff_h.at[c, pl.ds(base, CH)], srco.at[b], semi[b])
            pltpu.async_copy(dst_h.at[pl.ds(base, CH)], dsti.at[b], semi[b])
            pltpu.async_copy(pv_h.at[c, pl.ds(2 * base, 2 * CH)], pvf.at[b],
                             semi[b])

        def wait_idx(b, kk):
            base = (cbase + kk) * CH
            pltpu.make_async_copy(soff_h.at[c, pl.ds(base, CH)], srco.at[b],
                                  semi[b]).wait()
            pltpu.make_async_copy(dst_h.at[pl.ds(base, CH)], dsti.at[b],
                                  semi[b]).wait()
            pltpu.make_async_copy(pv_h.at[c, pl.ds(2 * base, 2 * CH)],
                                  pvf.at[b], semi[b]).wait()

        def fire(b, kk):
            wait_idx(b, kk)
            pltpu.async_copy(xt_h.at[srco.at[b]], xlb.at[b], sems[b])

        def drain(b):
            pltpu.make_async_copy(xt_h.at[srco.at[b]], xlb.at[b], sems[b]).wait()

        def compute(b):
            xl2d = xlb.at[b]
            ob2d = ob.at[b]
            pvfl = pvf.at[b]

            @pl.loop(0, CH, step=8)
            def _grp(g):
                pvec = pvfl[pl.ds(2 * g, 16)]
                for u in range(8):
                    e = g + u
                    xrow = xl2d.at[e]
                    orow = ob2d.at[e]
                    p0 = jnp.full((16,), pvec[2 * u], jnp.float32)
                    p1 = jnp.full((16,), pvec[2 * u + 1], jnp.float32)
                    orow[pl.ds(0, 16)] = p0 * xrow[pl.ds(0, 16)]
                    orow[pl.ds(16, 16)] = p1 * xrow[pl.ds(16, 16)]

            pltpu.sync_copy(ob2d, accs.at[dsti.at[b]], add=True)

        fire_idx(0, 0)
        fire(0, 0)
        fire_idx(1, 1)

        @pl.loop(0, NCHUNK, step=2)
        def _pair(kk):
            drain(0)
            fire(1, kk + 1)
            compute(0)
            fire_idx(0, jnp.minimum(kk + 2, NCHUNK - 1))
            drain(1)
            fire(0, jnp.minimum(kk + 2, NCHUNK - 1))
            compute(1)
            fire_idx(1, jnp.minimum(kk + 3, NCHUNK - 1))

        drain(0)
        wait_idx(1, NCHUNK - 1)
        plsc.subcore_barrier()
        pltpu.sync_copy(accs.at[pl.ds(s * RPT, RPT)],
                        acc_out.at[c, pl.ds(s * RPT, RPT)])

    return k(xtab, soff, dst, pvals, zat)


def _mk_tables(v):
    a = jnp.pad(v[:, :32], ((0, NP - N), (0, 0)))
    b = jnp.pad(v[:, 32:], ((0, NP - N), (0, 0)))
    return jnp.concatenate([a, b], axis=0)


def kernel(x, edge_index, hidden_state, enc_W1, enc_b1, enc_W2, enc_b2,
           g1_Wl, g1_bl, g1_Wr, g1_br, g1_att, g1_bias,
           g2_Wl, g2_bl, g2_Wr, g2_br, g2_att, g2_bias,
           gru_Wih, gru_bih, gru_Whh, gru_bhh,
           dec_W1, dec_b1, dec_W2, dec_b2):
    r1 = lambda v: v.reshape(1, -1)
    ar = jnp.arange(N, dtype=jnp.int32)
    padi = jnp.full((EP - E2,), N, jnp.int32)
    src = jnp.concatenate([edge_index[0], ar, padi])
    dst = jnp.concatenate([edge_index[1], ar, padi])
    soff = jnp.stack([src, src + NP])
    doff = jnp.stack([dst, dst + NP])
    zat = jnp.zeros((RPT, ACCW), jnp.float32)
    zp = jnp.zeros((RPT, 8), jnp.float32)

    xl1, xr1 = _tc_encode(x, enc_W1, r1(enc_b1), enc_W2, r1(enc_b2),
                          g1_Wl, r1(g1_bl), g1_Wr, r1(g1_br))
    xt1 = _mk_tables(xl1)
    pac1, pv1 = _sc_den(xt1, _mk_tables(xr1), g1_att, soff, doff, dst, zp)
    acc1 = _sc_agg(xt1, soff, dst, pv1, zat)
    xl2, xr2 = _tc_mid(acc1[0, :N], acc1[1, :N],
                       pac1[0, :N], pac1[1, :N], r1(g1_bias),
                       g2_Wl, r1(g2_bl), g2_Wr, r1(g2_br))
    xt2 = _mk_tables(xl2)
    pac2, pv2 = _sc_den(xt2, _mk_tables(xr2), g2_att, soff, doff, dst, zp)
    acc2 = _sc_agg(xt2, soff, dst, pv2, zat)
    out, new_hidden = _tc_final(acc2[0, :N], acc2[1, :N],
                                pac2[0, :N], pac2[1, :N], r1(g2_bias),
                                hidden_state, gru_Wih, r1(gru_bih),
                                gru_Whh, r1(gru_bhh),
                                dec_W1, r1(dec_b1), dec_W2, r1(dec_b2))
    return (out, new_hidden)


# speedup vs baseline: 2.2580x; 1.0769x over previous
import dataclasses
import functools

import jax
import jax.numpy as jnp
from jax import lax
from jax.experimental import pallas as pl
from jax.experimental.pallas import tpu as pltpu
from jax.experimental.pallas import tpu_sc as plsc

N = 50000
E = 800000
HID = 64
HEADS = 4
DOUT = 7

NC = 2
NT = 16
LANES = 16

NP = 50016
RPT = NP // NT
CH = 128
E2 = E + N
NCHUNK = 416
EPT = NCHUNK * CH
EP = EPT * NT
ACCW = 32

_ROWS = 2000
_GRID = N // _ROWS


def _tc_encode_body(x_ref, w1, b1, w2, b2, wl, bl, wr, br, xl_ref, xr_ref):
    h = jnp.maximum(jnp.dot(x_ref[...], w1[...],
                            preferred_element_type=jnp.float32) + b1[...], 0.0)
    h = jnp.dot(h, w2[...], preferred_element_type=jnp.float32) + b2[...]
    xl_ref[...] = jnp.dot(h, wl[...], preferred_element_type=jnp.float32) + bl[...]
    xr_ref[...] = jnp.dot(h, wr[...], preferred_element_type=jnp.float32) + br[...]


def _norm_heads(a, b, pa, pb):
    h0 = a[:, 0:16] / (pa[:, 0:1] + 1e-16)
    h1 = a[:, 16:32] / (pa[:, 1:2] + 1e-16)
    h2 = b[:, 0:16] / (pb[:, 0:1] + 1e-16)
    h3 = b[:, 16:32] / (pb[:, 1:2] + 1e-16)
    return jnp.concatenate([h0, h1, h2, h3], axis=1)


def _tc_mid_body(a_ref, b_ref, pa_ref, pb_ref, bias, wl, bl, wr, br,
                 xl_ref, xr_ref):
    h = jnp.maximum(_norm_heads(a_ref[...], b_ref[...], pa_ref[...],
                                pb_ref[...]) + bias[...], 0.0)
    xl_ref[...] = jnp.dot(h, wl[...], preferred_element_type=jnp.float32) + bl[...]
    xr_ref[...] = jnp.dot(h, wr[...], preferred_element_type=jnp.float32) + br[...]


def _tc_final_body(a_ref, b_ref, pa_ref, pb_ref, bias, hs_ref, wih, bih,
                   whh, bhh, dw1, db1, dw2, db2, out_ref, nh_ref):
    h = _norm_heads(a_ref[...], b_ref[...], pa_ref[...],
                    pb_ref[...]) + bias[...]
    hs = hs_ref[...]
    gi = jnp.dot(h, wih[...], preferred_element_type=jnp.float32) + bih[...]
    gh = jnp.dot(hs, whh[...], preferred_element_type=jnp.float32) + bhh[...]
    r = jax.nn.sigmoid(gi[:, 0:64] + gh[:, 0:64])
    z = jax.nn.sigmoid(gi[:, 64:128] + gh[:, 64:128])
    n = jnp.tanh(gi[:, 128:192] + r * gh[:, 128:192])
    nh = (1.0 - z) * n + z * hs
    d = jnp.maximum(jnp.dot(nh, dw1[...], preferred_element_type=jnp.float32)
                    + db1[...], 0.0)
    out_ref[...] = jnp.dot(d, dw2[...], preferred_element_type=jnp.float32) + db2[...]
    nh_ref[...] = nh


def _full(shape):
    return pl.BlockSpec(shape, lambda i: tuple(0 for _ in shape))


def _rows(width):
    return pl.BlockSpec((_ROWS, width), lambda i: (i, 0))


def _tc_encode(x, w1, b1, w2, b2, wl, bl, wr, br):
    return pl.pallas_call(
        _tc_encode_body,
        grid=(_GRID,),
        in_specs=[_rows(8), _full((8, 64)), _full((1, 64)), _full((64, 64)),
                  _full((1, 64)), _full((64, 64)), _full((1, 64)),
                  _full((64, 64)), _full((1, 64))],
        out_specs=[_rows(64), _rows(64)],
        out_shape=[jax.ShapeDtypeStruct((N, 64), jnp.float32),
                   jax.ShapeDtypeStruct((N, 64), jnp.float32)],
    )(x, w1, b1, w2, b2, wl, bl, wr, br)


def _tc_mid(a, b, pa, pb, bias, wl, bl, wr, br):
    return pl.pallas_call(
        _tc_mid_body,
        grid=(_GRID,),
        in_specs=[_rows(ACCW), _rows(ACCW), _rows(8), _rows(8),
                  _full((1, 64)), _full((64, 64)),
                  _full((1, 64)), _full((64, 64)), _full((1, 64))],
        out_specs=[_rows(64), _rows(64)],
        out_shape=[jax.ShapeDtypeStruct((N, 64), jnp.float32),
                   jax.ShapeDtypeStruct((N, 64), jnp.float32)],
    )(a, b, pa, pb, bias, wl, bl, wr, br)


def _tc_final(a, b, pa, pb, bias, hs, wih, bih, whh, bhh, dw1, db1, dw2, db2):
    return pl.pallas_call(
        _tc_final_body,
        grid=(_GRID,),
        in_specs=[_rows(ACCW), _rows(ACCW), _rows(8), _rows(8),
                  _full((1, 64)), _rows(64),
                  _full((64, 192)), _full((1, 192)), _full((64, 192)),
                  _full((1, 192)), _full((64, 64)), _full((1, 64)),
                  _full((64, 7)), _full((1, 7))],
        out_specs=[_rows(7), _rows(64)],
        out_shape=[jax.ShapeDtypeStruct((N, 7), jnp.float32),
                   jax.ShapeDtypeStruct((N, 64), jnp.float32)],
    )(a, b, pa, pb, bias, hs, wih, bih, whh, bhh, dw1, db1, dw2, db2)


def _sc_mesh_cp():
    mesh = plsc.VectorSubcoreMesh(core_axis_name="c", subcore_axis_name="s")
    cp = pltpu.CompilerParams()
    for fld, val in (("needs_layout_passes", False),
                     ("use_tc_tiling_on_sc", False)):
        if fld in pltpu.CompilerParams.__dataclass_fields__:
            cp = dataclasses.replace(cp, **{fld: val})
    return mesh, cp


def _sc_den(xtab, rtab, att, soff, doff, dst, zp):
    mesh, cp = _sc_mesh_cp()

    @functools.partial(
        pl.kernel,
        mesh=mesh,
        compiler_params=cp,
        out_type=(jax.ShapeDtypeStruct((NC, NP, 8), jnp.float32),
                  jax.ShapeDtypeStruct((NC, 2 * EP), jnp.float32)),
        scratch_types=[
            pltpu.VMEM((4, CH), jnp.int32),
            pltpu.VMEM((4, CH), jnp.int32),
            pltpu.VMEM((4, CH), jnp.int32),
            pltpu.VMEM((4, CH, 32), jnp.float32),
            pltpu.VMEM((4, CH, 32), jnp.float32),
            pltpu.VMEM((4, CH, 8), jnp.float32),
            pltpu.VMEM((4, 2 * CH), jnp.float32),
            pltpu.VMEM((2, 16), jnp.float32),
            pltpu.SemaphoreType.DMA,
            pltpu.SemaphoreType.DMA,
            pltpu.SemaphoreType.DMA,
            pltpu.SemaphoreType.DMA,
            pltpu.SemaphoreType.DMA,
            pltpu.SemaphoreType.DMA,
            pltpu.SemaphoreType.DMA,
            pltpu.SemaphoreType.DMA,
            pltpu.VMEM_SHARED((NP, 8), jnp.float32),
        ],
    )
    def k(xt_h, rt_h, att_h, soff_h, doff_h, dst_h, z_h, pacc_out, pv_out,
          dsti, srco, dsto, xlb, xrb, pb8, pvf, attb, sem0, sem1, sem2, sem3,
          semi0, semi1, semi2, semi3, accp):
        c = lax.axis_index("c")
        s = lax.axis_index("s")
        pltpu.sync_copy(att_h.at[pl.ds(2 * c, 2)], attb)
        pltpu.sync_copy(z_h, accp.at[pl.ds(s * RPT, RPT)])
        plsc.subcore_barrier()
        lid = lax.iota(jnp.int32, 16)
        att0 = attb[0]
        att1 = attb[1]
        sems = (sem0, sem1, sem2, sem3)
        cbase = s * NCHUNK
        perms = [((lid + sh) & 15).reshape(16, 1) for sh in (8, 4, 2, 1)]
        gdn = lax.GatherDimensionNumbers(offset_dims=(),
                                         collapsed_slice_dims=(0,),
                                         start_index_map=(0,))

        def _allsum(v):
            for pm in perms:
                v = v + lax.gather(v, pm, gdn, slice_sizes=(1,),
                                   mode=lax.GatherScatterMode.PROMISE_IN_BOUNDS)
            return v

        semi = (semi0, semi1, semi2, semi3)

        def fire_idx(b, kk):
            base = (cbase + kk) * CH
            pltpu.async_copy(soff_h.at[c, pl.ds(base, CH)], srco.at[b], semi[b])
            pltpu.async_copy(doff_h.at[c, pl.ds(base, CH)], dsto.at[b], semi[b])
            pltpu.async_copy(dst_h.at[pl.ds(base, CH)], dsti.at[b], semi[b])

        def wait_idx(b, kk):
            base = (cbase + kk) * CH
            pltpu.make_async_copy(soff_h.at[c, pl.ds(base, CH)], srco.at[b],
                                  semi[b]).wait()
            pltpu.make_async_copy(doff_h.at[c, pl.ds(base, CH)], dsto.at[b],
                                  semi[b]).wait()
            pltpu.make_async_copy(dst_h.at[pl.ds(base, CH)], dsti.at[b],
                                  semi[b]).wait()

        def fire(b, kk):
            wait_idx(b, kk)
            pltpu.async_copy(xt_h.at[srco.at[b]], xlb.at[b], sems[b])
            pltpu.async_copy(rt_h.at[dsto.at[b]], xrb.at[b], sems[b])

        def drain(b):
            pltpu.make_async_copy(xt_h.at[srco.at[b]], xlb.at[b], sems[b]).wait()
            pltpu.make_async_copy(rt_h.at[dsto.at[b]], xrb.at[b], sems[b]).wait()

        def compute(b, kk):
            xl2d = xlb.at[b]
            xr2d = xrb.at[b]
            pb2d = pb8.at[b]
            pvfl = pvf.at[b]

            @pl.loop(0, CH, step=LANES)
            def _grp(g):
                for u in range(LANES):
                    e = g + u
                    xrow = xl2d.at[e]
                    rrow = xr2d.at[e]
                    t0 = xrow[pl.ds(0, 16)] + rrow[pl.ds(0, 16)]
                    t1 = xrow[pl.ds(16, 16)] + rrow[pl.ds(16, 16)]
                    t0 = jnp.maximum(t0, 0.2 * t0)
                    t1 = jnp.maximum(t1, 0.2 * t1)
                    p0 = jnp.exp(_allsum(t0 * att0))
                    p1 = jnp.exp(_allsum(t1 * att1))
                    pb = jnp.where(lid == 0, p0, jnp.where(lid == 1, p1, 0.0))
                    plsc.store_scatter(pb2d.at[e], [lid], pb, mask=lid < 8)
                    plsc.store_scatter(pvfl, [2 * e + lid], pb, mask=lid < 2)

            base = (cbase + kk) * CH
            pltpu.sync_copy(pvfl, pv_out.at[c, pl.ds(2 * base, 2 * CH)])
            pltpu.sync_copy(pb2d, accp.at[dsti.at[b]], add=True)

        fire_idx(0, 0)
        fire_idx(1, 1)
        fire_idx(2, 2)
        fire(0, 0)
        fire(1, 1)
        fire(2, 2)
        fire_idx(3, 3)

        @pl.loop(0, NCHUNK, step=4)
        def _quad(kk):
            for d in range(4):
                drain(d)
                compute(d, kk + d)
                fire_idx(d, jnp.minimum(kk + d + 4, NCHUNK - 1))
                fire((d + 3) % 4, jnp.minimum(kk + d + 3, NCHUNK - 1))

        drain(0)
        drain(1)
        drain(2)
        wait_idx(3, NCHUNK - 1)
        plsc.subcore_barrier()
        pltpu.sync_copy(accp.at[pl.ds(s * RPT, RPT)],
                        pacc_out.at[c, pl.ds(s * RPT, RPT)])

    return k(xtab, rtab, att, soff, doff, dst, zp)


def _sc_agg(xtab, soff, dst, pvals, zat):
    mesh, cp = _sc_mesh_cp()

    @functools.partial(
        pl.kernel,
        mesh=mesh,
        compiler_params=cp,
        out_type=jax.ShapeDtypeStruct((NC, NP, ACCW), jnp.float32),
        scratch_types=[
            pltpu.VMEM((4, CH), jnp.int32),
            pltpu.VMEM((4, CH), jnp.int32),
            pltpu.VMEM((4, CH, 32), jnp.float32),
            pltpu.VMEM((4, 2 * CH), jnp.float32),
            pltpu.VMEM((CH, ACCW), jnp.float32),
            pltpu.SemaphoreType.DMA,
            pltpu.SemaphoreType.DMA,
            pltpu.SemaphoreType.DMA,
            pltpu.SemaphoreType.DMA,
            pltpu.SemaphoreType.DMA,
            pltpu.SemaphoreType.DMA,
            pltpu.SemaphoreType.DMA,
            pltpu.SemaphoreType.DMA,
            pltpu.VMEM_SHARED((NP, ACCW), jnp.float32),
        ],
    )
    def k(xt_h, soff_h, dst_h, pv_h, z_h, acc_out,
          dsti, srco, xlb, pvf, ob, sem0, sem1, sem2, sem3,
          semi0, semi1, semi2, semi3, accs):
        c = lax.axis_index("c")
        s = lax.axis_index("s")
        pltpu.sync_copy(z_h, accs.at[pl.ds(s * RPT, RPT)])
        plsc.subcore_barrier()
        sems = (sem0, sem1, sem2, sem3)
        semi = (semi0, semi1, semi2, semi3)
        cbase = s * NCHUNK

        def fire_idx(b, kk):
            base = (cbase + kk) * CH
            pltpu.async_copy(soff_h.at[c, pl.ds(base, CH)], srco.at[b], semi[b])
            pltpu.async_copy(dst_h.at[pl.ds(base, CH)], dsti.at[b], semi[b])
            pltpu.async_copy(pv_h.at[c, pl.ds(2 * base, 2 * CH)], pvf.at[b],
                             semi[b])

        def wait_idx(b, kk):
            base = (cbase + kk) * CH
            pltpu.make_async_copy(soff_h.at[c, pl.ds(base, CH)], srco.at[b],
                                  semi[b]).wait()
            pltpu.make_async_copy(dst_h.at[pl.ds(base, CH)], dsti.at[b],
                                  semi[b]).wait()
            pltpu.make_async_copy(pv_h.at[c, pl.ds(2 * base, 2 * CH)],
                                  pvf.at[b], semi[b]).wait()

        def fire(b, kk):
            wait_idx(b, kk)
            pltpu.async_copy(xt_h.at[srco.at[b]], xlb.at[b], sems[b])

        def drain(b):
            pltpu.make_async_copy(xt_h.at[srco.at[b]], xlb.at[b], sems[b]).wait()

        def compute(b):
            xl2d = xlb.at[b]
            pvfl = pvf.at[b]

            @pl.loop(0, CH, step=8)
            def _grp(g):
                pvec = pvfl[pl.ds(2 * g, 16)]
                for u in range(8):
                    e = g + u
                    xrow = xl2d.at[e]
                    orow = ob.at[e]
                    p0 = jnp.full((16,), pvec[2 * u], jnp.float32)
                    p1 = jnp.full((16,), pvec[2 * u + 1], jnp.float32)
                    orow[pl.ds(0, 16)] = p0 * xrow[pl.ds(0, 16)]
                    orow[pl.ds(16, 16)] = p1 * xrow[pl.ds(16, 16)]

            pltpu.sync_copy(ob, accs.at[dsti.at[b]], add=True)

        fire_idx(0, 0)
        fire_idx(1, 1)
        fire_idx(2, 2)
        fire(0, 0)
        fire(1, 1)
        fire(2, 2)
        fire_idx(3, 3)

        @pl.loop(0, NCHUNK, step=4)
        def _quad(kk):
            for d in range(4):
                drain(d)
                compute(d)
                fire_idx(d, jnp.minimum(kk + d + 4, NCHUNK - 1))
                fire((d + 3) % 4, jnp.minimum(kk + d + 3, NCHUNK - 1))

        drain(0)
        drain(1)
        drain(2)
        wait_idx(3, NCHUNK - 1)
        plsc.subcore_barrier()
        pltpu.sync_copy(accs.at[pl.ds(s * RPT, RPT)],
                        acc_out.at[c, pl.ds(s * RPT, RPT)])

    return k(xtab, soff, dst, pvals, zat)


def _mk_tables(v):
    a = jnp.pad(v[:, :32], ((0, NP - N), (0, 0)))
    b = jnp.pad(v[:, 32:], ((0, NP - N), (0, 0)))
    return jnp.concatenate([a, b], axis=0)


def kernel(x, edge_index, hidden_state, enc_W1, enc_b1, enc_W2, enc_b2,
           g1_Wl, g1_bl, g1_Wr, g1_br, g1_att, g1_bias,
           g2_Wl, g2_bl, g2_Wr, g2_br, g2_att, g2_bias,
           gru_Wih, gru_bih, gru_Whh, gru_bhh,
           dec_W1, dec_b1, dec_W2, dec_b2):
    r1 = lambda v: v.reshape(1, -1)
    ar = jnp.arange(N, dtype=jnp.int32)
    padi = jnp.full((EP - E2,), N, jnp.int32)
    src = jnp.concatenate([edge_index[0], ar, padi])
    dst = jnp.concatenate([edge_index[1], ar, padi])
    soff = jnp.stack([src, src + NP])
    doff = jnp.stack([dst, dst + NP])
    zat = jnp.zeros((RPT, ACCW), jnp.float32)
    zp = jnp.zeros((RPT, 8), jnp.float32)

    xl1, xr1 = _tc_encode(x, enc_W1, r1(enc_b1), enc_W2, r1(enc_b2),
                          g1_Wl, r1(g1_bl), g1_Wr, r1(g1_br))
    xt1 = _mk_tables(xl1)
    pac1, pv1 = _sc_den(xt1, _mk_tables(xr1), g1_att, soff, doff, dst, zp)
    acc1 = _sc_agg(xt1, soff, dst, pv1, zat)
    xl2, xr2 = _tc_mid(acc1[0, :N], acc1[1, :N],
                       pac1[0, :N], pac1[1, :N], r1(g1_bias),
                       g2_Wl, r1(g2_bl), g2_Wr, r1(g2_br))
    xt2 = _mk_tables(xl2)
    pac2, pv2 = _sc_den(xt2, _mk_tables(xr2), g2_att, soff, doff, dst, zp)
    acc2 = _sc_agg(xt2, soff, dst, pv2, zat)
    out, new_hidden = _tc_final(acc2[0, :N], acc2[1, :N],
                                pac2[0, :N], pac2[1, :N], r1(g2_bias),
                                hidden_state, gru_Wih, r1(gru_bih),
                                gru_Whh, r1(gru_bhh),
                                dec_W1, r1(dec_b1), dec_W2, r1(dec_b2))
    return (out, new_hidden)
